# baseline (speedup 1.0000x reference)
import functools

import jax
import jax.numpy as jnp
from jax import lax
from jax.experimental import pallas as pl
from jax.experimental.pallas import tpu as pltpu
from jax.experimental.pallas import tpu_sc as plsc

D_MODEL = 64
SCALE = float(D_MODEL) ** 0.5

_INFO = plsc.get_sparse_core_info()
_NC = _INFO.num_cores
_NS = _INFO.num_subcores
_NW = _NC * _NS
_LANES = _INFO.num_lanes

_IW = 128
_GRP = 2
_CHUNK = _GRP * _IW
_STAGE = 1024
_TPC = _CHUNK // 8


@functools.partial(jax.jit, static_argnames=("n_rows",))
def _embed(x1, table, n_rows):
    rows_per_w = n_rows // _NW
    chunks = rows_per_w // _STAGE

    mesh = plsc.VectorSubcoreMesh(core_axis_name="c", subcore_axis_name="s")

    @functools.partial(
        pl.kernel,
        mesh=mesh,
        out_type=jax.ShapeDtypeStruct((n_rows // 8, 8, 2 * D_MODEL),
                                      jnp.float32),
        scratch_types=[
            pltpu.VMEM((_STAGE,), jnp.int32),
            pltpu.VMEM((_CHUNK, D_MODEL), jnp.float32),
            pltpu.VMEM((_TPC, 8, 2 * D_MODEL), jnp.float32),
            pltpu.SemaphoreType.DMA,
        ],
        compiler_params=pltpu.CompilerParams(use_tc_tiling_on_sc=False),
    )
    def k(x_hbm, table_hbm, out_hbm, idx_v, rows_v, pack_v, gsem):
        wid = lax.axis_index("s") * _NC + lax.axis_index("c")
        base = wid * rows_per_w
        tile_base = base // 8

        def chunk_body(t, _):
            pltpu.sync_copy(x_hbm.at[pl.ds(base + t * _STAGE, _STAGE)], idx_v)
            for s in range(_STAGE // _CHUNK):
                descs = []
                for j in range(_GRP):
                    descs.append(
                        pltpu.async_copy(
                            table_hbm.at[
                                idx_v.at[pl.ds((s * _GRP + j) * _IW, _IW)]
                            ],
                            rows_v.at[pl.ds(j * _IW, _IW)],
                            gsem,
                        )
                    )
                for d in descs:
                    d.wait()

                def pack_body(r, _):
                    tb = r // 8
                    rr = r % 8
                    for c in range(D_MODEL // _LANES):
                        sl = pl.ds(c * _LANES, _LANES)
                        pack_v[tb, rr, sl] = rows_v[r, sl] * SCALE
                    return ()

                lax.fori_loop(0, _CHUNK, pack_body, ())

                pltpu.sync_copy(
                    pack_v,
                    out_hbm.at[
                        pl.ds(tile_base + (t * (_STAGE // _CHUNK) + s) * _TPC,
                              _TPC)
                    ],
                )
            return ()

        lax.fori_loop(0, chunks, chunk_body, ())

    return k(x1, table)


def kernel(x, table):
    b0, b1 = x.shape
    n_rows = b0 * b1
    x1 = x.reshape(n_rows).astype(jnp.int32)
    out = _embed(x1, table, n_rows)
    return out[:, :, :D_MODEL].reshape(n_rows, D_MODEL).reshape(
        b0, b1, D_MODEL)


# speedup vs baseline: 1.0109x; 1.0109x over previous
import functools

import jax
import jax.numpy as jnp
from jax import lax
from jax.experimental import pallas as pl
from jax.experimental.pallas import tpu as pltpu
from jax.experimental.pallas import tpu_sc as plsc

D_MODEL = 64
SCALE = float(D_MODEL) ** 0.5

_INFO = plsc.get_sparse_core_info()
_NC = _INFO.num_cores
_NS = _INFO.num_subcores
_NW = _NC * _NS
_LANES = _INFO.num_lanes

_IW = 128
_GRP = 2
_CHUNK = _GRP * _IW
_STAGE = 1024
_TPC = _CHUNK // 8


@functools.partial(jax.jit, static_argnames=("n_rows",))
def _embed(x1, table, n_rows):
    rows_per_w = n_rows // _NW
    chunks = rows_per_w // _STAGE

    mesh = plsc.VectorSubcoreMesh(core_axis_name="c", subcore_axis_name="s")

    @functools.partial(
        pl.kernel,
        mesh=mesh,
        out_type=jax.ShapeDtypeStruct((n_rows // 8, 8, 2 * D_MODEL),
                                      jnp.float32),
        scratch_types=[
            pltpu.VMEM((_STAGE,), jnp.int32),
            pltpu.VMEM((_CHUNK, D_MODEL), jnp.float32),
            pltpu.VMEM((_TPC, 8, 2 * D_MODEL), jnp.float32),
            pltpu.SemaphoreType.DMA,
        ],
        compiler_params=pltpu.CompilerParams(use_tc_tiling_on_sc=False),
    )
    def k(x_hbm, table_hbm, out_hbm, idx_v, rows_v, pack_v, gsem):
        wid = lax.axis_index("s") * _NC + lax.axis_index("c")
        base = wid * rows_per_w
        tile_base = base // 8

        def chunk_body(t, _):
            pltpu.sync_copy(x_hbm.at[pl.ds(base + t * _STAGE, _STAGE)], idx_v)
            for s in range(_STAGE // _CHUNK):
                descs = []
                for j in range(_GRP):
                    descs.append(
                        pltpu.async_copy(
                            table_hbm.at[
                                idx_v.at[pl.ds((s * _GRP + j) * _IW, _IW)]
                            ],
                            rows_v.at[pl.ds(j * _IW, _IW)],
                            gsem,
                        )
                    )
                for d in descs:
                    d.wait()

                def pack_body(tb, _):
                    r0 = tb * 8
                    for rr in range(8):
                        for c in range(D_MODEL // _LANES):
                            sl = pl.ds(c * _LANES, _LANES)
                            pack_v[tb, rr, sl] = rows_v[r0 + rr, sl] * SCALE
                    return ()

                lax.fori_loop(0, _TPC, pack_body, ())

                pltpu.sync_copy(
                    pack_v,
                    out_hbm.at[
                        pl.ds(tile_base + (t * (_STAGE // _CHUNK) + s) * _TPC,
                              _TPC)
                    ],
                )
            return ()

        lax.fori_loop(0, chunks, chunk_body, ())

    return k(x1, table)


def kernel(x, table):
    b0, b1 = x.shape
    n_rows = b0 * b1
    x1 = x.reshape(n_rows).astype(jnp.int32)
    out = _embed(x1, table, n_rows)
    return out[:, :, :D_MODEL].reshape(n_rows, D_MODEL).reshape(
        b0, b1, D_MODEL)
